# baseline (speedup 1.0000x reference)
import jax
import jax.numpy as jnp
from jax import lax
from jax.experimental import pallas as pl
from jax.experimental.pallas import tpu as pltpu
from jax.experimental.pallas import tpu_sc as plsc

_N = 100000
_E = 1600000
_K = 32
_KH = 16
_NS = 16
_SUB = 128
_NB = _E // _SUB
_CH = 2
_NIT = _NB // _CH
_R_MAIN = 6256
_R_LAST = _N - 15 * _R_MAIN


def _sc_body(t0t, t1lo, t1hi, opf_t, out0t, out1,
             idxb, t0Ts, t0b, t1b, acc,
             sem_in, sem_g0, sem_g1, sem_sc, sem_out):
    cid = lax.axis_index("c")
    wid = lax.axis_index("s")

    iota = lax.iota(jnp.int32, 16)
    gg_idx = lax.shift_right_logical(iota, 3)
    r8_idx = lax.bitwise_and(iota, 7)

    def for_my_half(fn):
        @pl.when(cid == 0)
        def _():
            fn(t1lo, 0, 0)

        @pl.when(cid == 1)
        def _():
            fn(t1hi, _KH, 2)

    def per_tile_rows(fn):
        rbase = wid * _R_MAIN

        @pl.when(wid < _NS - 1)
        def _():
            fn(rbase, _R_MAIN)

        @pl.when(wid == _NS - 1)
        def _():
            fn(rbase, _R_LAST)

    def init(t1h, koff, g0):
        def cp(rbase, nrows):
            pltpu.sync_copy(t1h.at[pl.ds(rbase, nrows)],
                            acc.at[pl.ds(rbase, nrows)])
        per_tile_rows(cp)

    for_my_half(init)
    plsc.subcore_barrier()

    lo = (wid * _NIT) // _NS
    hi = ((wid + 1) * _NIT) // _NS

    def run_half(t1h, koff, g0):
        def issue_in(it, s):
            b = it * _CH
            pltpu.async_copy(opf_t.at[pl.ds(b, _CH)], idxb.at[s], sem_in)
            pltpu.async_copy(t0t.at[pl.ds(g0, 2), pl.ds(b, _CH)],
                             t0Ts.at[s], sem_in)

        def wait_in():
            pltpu.make_async_copy(opf_t.at[pl.ds(0, _CH)], idxb.at[0],
                                  sem_in).wait()
            pltpu.make_async_copy(t0t.at[pl.ds(0, 2), pl.ds(0, _CH)],
                                  t0Ts.at[0], sem_in).wait()

        def issue_gather_p(sp):
            sem = sem_g0 if sp == 0 else sem_g1
            for j in range(_CH):
                pltpu.async_copy(t1h.at[idxb.at[sp, j, 1]],
                                 t1b.at[sp, pl.ds(j * _SUB, _SUB)], sem)

        def wait_gather_p(sp):
            sem = sem_g0 if sp == 0 else sem_g1
            for j in range(_CH):
                pltpu.make_async_copy(
                    t1h.at[idxb.at[0, j, 1]],
                    t1b.at[0, pl.ds(j * _SUB, _SUB)], sem).wait()

        def issue_gather(sdyn):
            @pl.when(sdyn == 0)
            def _():
                issue_gather_p(0)

            @pl.when(sdyn == 1)
            def _():
                issue_gather_p(1)

        def wait_gather(sdyn):
            @pl.when(sdyn == 0)
            def _():
                wait_gather_p(0)

            @pl.when(sdyn == 1)
            def _():
                wait_gather_p(1)

        def wait_scatter():
            for j in range(_CH):
                pltpu.make_async_copy(
                    t0b.at[0, pl.ds(j * _SUB, _SUB)],
                    acc.at[idxb.at[0, j, 1]], sem_sc).wait()

        def wait_out():
            pltpu.make_async_copy(t0Ts.at[0],
                                  out0t.at[pl.ds(0, 2), pl.ds(0, _CH)],
                                  sem_out).wait()

        issue_in(lo, lo & 1)
        wait_in()
        issue_gather(lo & 1)

        def step(it, carry):
            s = it & 1
            b = it * _CH

            @pl.when(it > lo)
            def _():
                wait_scatter()
                wait_out()

            @pl.when(it + 1 < hi)
            def _():
                issue_in(it + 1, s ^ 1)

            ssp = jnp.full((16,), s, dtype=jnp.int32)
            for j in range(_CH):
                jsp = jnp.full((16,), j, dtype=jnp.int32)

                @plsc.parallel_loop(0, _SUB, unroll=8)
                def _(c):
                    csp = jnp.full((16,), c, dtype=jnp.int32)
                    t0b[s, j * _SUB + c, :] = plsc.load_gather(
                        t0Ts, [ssp, gg_idx, jsp, r8_idx, csp])

            for j in range(_CH):
                pltpu.async_copy(t0b.at[s, pl.ds(j * _SUB, _SUB)],
                                 acc.at[idxb.at[s, j, 1]], sem_sc, add=True)

            @pl.when(it + 1 < hi)
            def _():
                wait_in()
                issue_gather(s ^ 1)

            wait_gather(s)

            for j in range(_CH):
                jsp = jnp.full((16,), j, dtype=jnp.int32)

                @plsc.parallel_loop(0, _SUB, unroll=8)
                def _(c):
                    csp = jnp.full((16,), c, dtype=jnp.int32)
                    plsc.addupdate_scatter(
                        t0Ts, [ssp, gg_idx, jsp, r8_idx, csp],
                        t1b[s, j * _SUB + c, :])

            pltpu.async_copy(t0Ts.at[s],
                             out0t.at[pl.ds(g0, 2), pl.ds(b, _CH)], sem_out)
            return carry

        lax.fori_loop(lo, hi, step, jnp.int32(0))
        wait_scatter()
        wait_out()

    for_my_half(run_half)

    plsc.subcore_barrier()

    def drain(t1h, koff, g0):
        def cp(rbase, nrows):
            pltpu.sync_copy(acc.at[pl.ds(rbase, nrows)],
                            out1.at[pl.ds(rbase, nrows), pl.ds(koff, _KH)])
        per_tile_rows(cp)

    for_my_half(drain)


def kernel(theta_0, theta_1, op_fwd, op_bwd):
    t0t = jnp.transpose(theta_0.reshape(_NB, _SUB, 4, 8), (2, 0, 3, 1))
    opf_t = jnp.transpose(op_fwd.reshape(2, _NB, _SUB), (1, 0, 2))
    t1lo = theta_1[:, :_KH]
    t1hi = theta_1[:, _KH:]

    mesh = plsc.VectorSubcoreMesh(core_axis_name="c", subcore_axis_name="s")
    out0t, out1 = pl.kernel(
        _sc_body,
        out_type=[
            jax.ShapeDtypeStruct((4, _NB, 8, _SUB), jnp.float32),
            jax.ShapeDtypeStruct((_N, _K), jnp.float32),
        ],
        mesh=mesh,
        compiler_params=pltpu.CompilerParams(
            use_tc_tiling_on_sc=False, needs_layout_passes=False),
        scratch_types=[
            pltpu.VMEM((2, _CH, 2, _SUB), jnp.int32),
            pltpu.VMEM((2, 2, _CH, 8, _SUB), jnp.float32),
            pltpu.VMEM((2, _CH * _SUB, _KH), jnp.float32),
            pltpu.VMEM((2, _CH * _SUB, _KH), jnp.float32),
            pltpu.VMEM_SHARED((_N, _KH), jnp.float32),
            pltpu.SemaphoreType.DMA,
            pltpu.SemaphoreType.DMA,
            pltpu.SemaphoreType.DMA,
            pltpu.SemaphoreType.DMA,
            pltpu.SemaphoreType.DMA,
        ],
    )(t0t, t1lo, t1hi, opf_t)

    out0 = jnp.transpose(out0t, (1, 3, 0, 2)).reshape(_E, _K)
    return out0, out1


# speedup vs baseline: 1.0185x; 1.0185x over previous
import jax
import jax.numpy as jnp
from jax import lax
from jax.experimental import pallas as pl
from jax.experimental.pallas import tpu as pltpu
from jax.experimental.pallas import tpu_sc as plsc

_N = 100000
_E = 1600000
_K = 32
_KH = 16
_NS = 16
_SUB = 128
_NB = _E // _SUB
_CH = 2
_NIT = _NB // _CH
_R_MAIN = 6256
_R_LAST = _N - 15 * _R_MAIN


def _sc_body(t0t, t1lo, t1hi, opf_t, out0t, out1,
             idxb, t0Ts, t0b, t1b, acc,
             sem_in, sem_g0, sem_g1, sem_sc, sem_out):
    cid = lax.axis_index("c")
    wid = lax.axis_index("s")

    iota = lax.iota(jnp.int32, 16)
    gg_idx = lax.shift_right_logical(iota, 3)
    r8_idx = lax.bitwise_and(iota, 7)
    tbase = (gg_idx * (1024 * _CH)) + (r8_idx * 128)

    def for_my_half(fn):
        @pl.when(cid == 0)
        def _():
            fn(t1lo, 0, 0)

        @pl.when(cid == 1)
        def _():
            fn(t1hi, _KH, 2)

    def per_tile_rows(fn):
        rbase = wid * _R_MAIN

        @pl.when(wid < _NS - 1)
        def _():
            fn(rbase, _R_MAIN)

        @pl.when(wid == _NS - 1)
        def _():
            fn(rbase, _R_LAST)

    def init(t1h, koff, g0):
        def cp(rbase, nrows):
            pltpu.sync_copy(t1h.at[pl.ds(rbase, nrows)],
                            acc.at[pl.ds(rbase, nrows)])
        per_tile_rows(cp)

    for_my_half(init)
    plsc.subcore_barrier()

    lo = (wid * _NIT) // _NS
    hi = ((wid + 1) * _NIT) // _NS

    def run_half(t1h, koff, g0):
        def issue_in(it, s):
            b = it * _CH
            pltpu.async_copy(opf_t.at[pl.ds(b, _CH)], idxb.at[s], sem_in)
            for gg in range(2):
                pltpu.async_copy(
                    t0t.at[pl.ds(((g0 + gg) * _NB + b) * 1024, _CH * 1024)],
                    t0Ts.at[pl.ds((s * 2 + gg) * _CH * 1024, _CH * 1024)],
                    sem_in)

        def wait_in():
            pltpu.make_async_copy(opf_t.at[pl.ds(0, _CH)], idxb.at[0],
                                  sem_in).wait()
            for gg in range(2):
                pltpu.make_async_copy(
                    t0t.at[pl.ds(0, _CH * 1024)],
                    t0Ts.at[pl.ds(0, _CH * 1024)], sem_in).wait()

        def issue_gather_p(sp):
            sem = sem_g0 if sp == 0 else sem_g1
            for j in range(_CH):
                pltpu.async_copy(t1h.at[idxb.at[sp, j, 1]],
                                 t1b.at[sp, pl.ds(j * _SUB, _SUB)], sem)

        def wait_gather_p(sp):
            sem = sem_g0 if sp == 0 else sem_g1
            for j in range(_CH):
                pltpu.make_async_copy(
                    t1h.at[idxb.at[0, j, 1]],
                    t1b.at[0, pl.ds(j * _SUB, _SUB)], sem).wait()

        def issue_gather(sdyn):
            @pl.when(sdyn == 0)
            def _():
                issue_gather_p(0)

            @pl.when(sdyn == 1)
            def _():
                issue_gather_p(1)

        def wait_gather(sdyn):
            @pl.when(sdyn == 0)
            def _():
                wait_gather_p(0)

            @pl.when(sdyn == 1)
            def _():
                wait_gather_p(1)

        def wait_scatter():
            for j in range(_CH):
                pltpu.make_async_copy(
                    t0b.at[0, pl.ds(j * _SUB, _SUB)],
                    acc.at[idxb.at[0, j, 1]], sem_sc).wait()

        def wait_out():
            for gg in range(2):
                pltpu.make_async_copy(
                    t0Ts.at[pl.ds(0, _CH * 1024)],
                    out0t.at[pl.ds(0, _CH * 1024)], sem_out).wait()

        issue_in(lo, lo & 1)
        wait_in()
        issue_gather(lo & 1)

        def step(it, carry):
            s = it & 1
            b = it * _CH

            @pl.when(it > lo)
            def _():
                wait_scatter()
                wait_out()

            @pl.when(it + 1 < hi)
            def _():
                issue_in(it + 1, s ^ 1)

            for j in range(_CH):
                a0 = tbase + (s * 2 * _CH * 1024 + j * 1024)

                @plsc.parallel_loop(0, _SUB, unroll=8, carry=a0)
                def _(c, idxv):
                    t0b[s, j * _SUB + c, :] = plsc.load_gather(t0Ts, [idxv])
                    return idxv + 1

            for j in range(_CH):
                pltpu.async_copy(t0b.at[s, pl.ds(j * _SUB, _SUB)],
                                 acc.at[idxb.at[s, j, 1]], sem_sc, add=True)

            @pl.when(it + 1 < hi)
            def _():
                wait_in()
                issue_gather(s ^ 1)

            wait_gather(s)

            for j in range(_CH):
                a0 = tbase + (s * 2 * _CH * 1024 + j * 1024)

                @plsc.parallel_loop(0, _SUB, unroll=8, carry=a0)
                def _(c, idxv):
                    plsc.addupdate_scatter(t0Ts, [idxv],
                                           t1b[s, j * _SUB + c, :])
                    return idxv + 1

            for gg in range(2):
                pltpu.async_copy(
                    t0Ts.at[pl.ds((s * 2 + gg) * _CH * 1024, _CH * 1024)],
                    out0t.at[pl.ds(((g0 + gg) * _NB + b) * 1024, _CH * 1024)],
                    sem_out)
            return carry

        lax.fori_loop(lo, hi, step, jnp.int32(0))
        wait_scatter()
        wait_out()

    for_my_half(run_half)

    plsc.subcore_barrier()

    def drain(t1h, koff, g0):
        def cp(rbase, nrows):
            pltpu.sync_copy(acc.at[pl.ds(rbase, nrows)],
                            out1.at[pl.ds(rbase, nrows), pl.ds(koff, _KH)])
        per_tile_rows(cp)

    for_my_half(drain)


def kernel(theta_0, theta_1, op_fwd, op_bwd):
    t0t = jnp.transpose(theta_0.reshape(_NB, _SUB, 4, 8),
                        (2, 0, 3, 1)).reshape(-1)
    opf_t = jnp.transpose(op_fwd.reshape(2, _NB, _SUB), (1, 0, 2))
    t1lo = theta_1[:, :_KH]
    t1hi = theta_1[:, _KH:]

    mesh = plsc.VectorSubcoreMesh(core_axis_name="c", subcore_axis_name="s")
    out0t, out1 = pl.kernel(
        _sc_body,
        out_type=[
            jax.ShapeDtypeStruct((4 * _NB * 8 * _SUB,), jnp.float32),
            jax.ShapeDtypeStruct((_N, _K), jnp.float32),
        ],
        mesh=mesh,
        compiler_params=pltpu.CompilerParams(
            use_tc_tiling_on_sc=False, needs_layout_passes=False),
        scratch_types=[
            pltpu.VMEM((2, _CH, 2, _SUB), jnp.int32),
            pltpu.VMEM((2 * 2 * _CH * 8 * _SUB,), jnp.float32),
            pltpu.VMEM((2, _CH * _SUB, _KH), jnp.float32),
            pltpu.VMEM((2, _CH * _SUB, _KH), jnp.float32),
            pltpu.VMEM_SHARED((_N, _KH), jnp.float32),
            pltpu.SemaphoreType.DMA,
            pltpu.SemaphoreType.DMA,
            pltpu.SemaphoreType.DMA,
            pltpu.SemaphoreType.DMA,
            pltpu.SemaphoreType.DMA,
        ],
    )(t0t, t1lo, t1hi, opf_t)

    out0 = jnp.transpose(out0t.reshape(4, _NB, 8, _SUB),
                         (1, 3, 0, 2)).reshape(_E, _K)
    return out0, out1


# speedup vs baseline: 2.0044x; 1.9680x over previous
import jax
import jax.numpy as jnp
from jax import lax
from jax.experimental import pallas as pl
from jax.experimental.pallas import tpu as pltpu
from jax.experimental.pallas import tpu_sc as plsc

_N = 100000
_E = 1600000
_K = 32
_KH = 16
_NS = 16
_SUB = 128
_NB = _E // _SUB
_CH = 2
_NIT = _NB // _CH
_RS = 2 * (_CH * 8) + 8
_PITCH = _SUB + 1
_R_MAIN = 6256
_R_LAST = _N - 15 * _R_MAIN


def _sc_body(t0t, t1lo, t1hi, opf_t, out0t, out1,
             idxb, t0Ts, t0b, t1b, acc,
             sem_in, sem_g0, sem_g1, sem_sc, sem_out):
    cid = lax.axis_index("c")
    wid = lax.axis_index("s")

    iota = lax.iota(jnp.int32, 16)
    gg_idx = lax.shift_right_logical(iota, 3)
    r8_idx = lax.bitwise_and(iota, 7)
    rowconst = gg_idx * (_CH * 8 + 8) + r8_idx

    def for_my_half(fn):
        @pl.when(cid == 0)
        def _():
            fn(t1lo, 0, 0)

        @pl.when(cid == 1)
        def _():
            fn(t1hi, _KH, 2)

    def per_tile_rows(fn):
        rbase = wid * _R_MAIN

        @pl.when(wid < _NS - 1)
        def _():
            fn(rbase, _R_MAIN)

        @pl.when(wid == _NS - 1)
        def _():
            fn(rbase, _R_LAST)

    def init(t1h, koff, g0):
        def cp(rbase, nrows):
            pltpu.sync_copy(t1h.at[pl.ds(rbase, nrows)],
                            acc.at[pl.ds(rbase, nrows)])
        per_tile_rows(cp)

    for_my_half(init)
    plsc.subcore_barrier()

    lo = (wid * _NIT) // _NS
    hi = ((wid + 1) * _NIT) // _NS

    def run_half(t1h, koff, g0):
        def issue_in(it, s):
            b = it * _CH
            pltpu.async_copy(opf_t.at[pl.ds(b, _CH)], idxb.at[s], sem_in)
            for gg in range(2):
                pltpu.async_copy(
                    t0t.at[pl.ds(((g0 + gg) * _NB + b) * 8, _CH * 8), :],
                    t0Ts.at[pl.ds(s * _RS + gg * (_CH * 8 + 8), _CH * 8),
                            pl.ds(0, _SUB)],
                    sem_in)

        def wait_in():
            pltpu.make_async_copy(opf_t.at[pl.ds(0, _CH)], idxb.at[0],
                                  sem_in).wait()
            for gg in range(2):
                pltpu.make_async_copy(
                    t0t.at[pl.ds(0, _CH * 8), :],
                    t0Ts.at[pl.ds(0, _CH * 8), pl.ds(0, _SUB)],
                    sem_in).wait()

        def issue_gather_p(sp):
            sem = sem_g0 if sp == 0 else sem_g1
            for j in range(_CH):
                pltpu.async_copy(t1h.at[idxb.at[sp, j, 1]],
                                 t1b.at[sp, pl.ds(j * _SUB, _SUB)], sem)

        def wait_gather_p(sp):
            sem = sem_g0 if sp == 0 else sem_g1
            for j in range(_CH):
                pltpu.make_async_copy(
                    t1h.at[idxb.at[0, j, 1]],
                    t1b.at[0, pl.ds(j * _SUB, _SUB)], sem).wait()

        def issue_gather(sdyn):
            @pl.when(sdyn == 0)
            def _():
                issue_gather_p(0)

            @pl.when(sdyn == 1)
            def _():
                issue_gather_p(1)

        def wait_gather(sdyn):
            @pl.when(sdyn == 0)
            def _():
                wait_gather_p(0)

            @pl.when(sdyn == 1)
            def _():
                wait_gather_p(1)

        def wait_scatter():
            for j in range(_CH):
                pltpu.make_async_copy(
                    t0b.at[0, pl.ds(j * _SUB, _SUB)],
                    acc.at[idxb.at[0, j, 1]], sem_sc).wait()

        def wait_out():
            for gg in range(2):
                pltpu.make_async_copy(
                    t0Ts.at[pl.ds(0, _CH * 8), pl.ds(0, _SUB)],
                    out0t.at[pl.ds(0, _CH * 8), :], sem_out).wait()

        issue_in(lo, lo & 1)
        wait_in()
        issue_gather(lo & 1)

        def step(it, carry):
            s = it & 1
            b = it * _CH

            @pl.when(it > lo)
            def _():
                wait_scatter()
                wait_out()

            @pl.when(it + 1 < hi)
            def _():
                issue_in(it + 1, s ^ 1)

            for j in range(_CH):
                rowv = rowconst + (s * _RS + j * 8)

                @plsc.parallel_loop(0, _SUB, unroll=8,
                                    carry=jnp.zeros((16,), jnp.int32))
                def _(c, colv):
                    t0b[s, j * _SUB + c, :] = plsc.load_gather(
                        t0Ts, [rowv, colv])
                    return colv + 1

            for j in range(_CH):
                pltpu.async_copy(t0b.at[s, pl.ds(j * _SUB, _SUB)],
                                 acc.at[idxb.at[s, j, 1]], sem_sc, add=True)

            @pl.when(it + 1 < hi)
            def _():
                wait_in()
                issue_gather(s ^ 1)

            wait_gather(s)

            for j in range(_CH):
                rowv = rowconst + (s * _RS + j * 8)

                @plsc.parallel_loop(0, _SUB, unroll=8,
                                    carry=jnp.zeros((16,), jnp.int32))
                def _(c, colv):
                    plsc.addupdate_scatter(t0Ts, [rowv, colv],
                                           t1b[s, j * _SUB + c, :])
                    return colv + 1

            for gg in range(2):
                pltpu.async_copy(
                    t0Ts.at[pl.ds(s * _RS + gg * (_CH * 8 + 8), _CH * 8),
                            pl.ds(0, _SUB)],
                    out0t.at[pl.ds(((g0 + gg) * _NB + b) * 8, _CH * 8), :],
                    sem_out)
            return carry

        lax.fori_loop(lo, hi, step, jnp.int32(0))
        wait_scatter()
        wait_out()

    for_my_half(run_half)

    plsc.subcore_barrier()

    def drain(t1h, koff, g0):
        def cp(rbase, nrows):
            pltpu.sync_copy(acc.at[pl.ds(rbase, nrows)],
                            out1.at[pl.ds(rbase, nrows), pl.ds(koff, _KH)])
        per_tile_rows(cp)

    for_my_half(drain)


def kernel(theta_0, theta_1, op_fwd, op_bwd):
    t0t = jnp.transpose(theta_0.reshape(_NB, _SUB, 4, 8),
                        (2, 0, 3, 1)).reshape(4 * _NB * 8, _SUB)
    opf_t = jnp.transpose(op_fwd.reshape(2, _NB, _SUB), (1, 0, 2))
    t1lo = theta_1[:, :_KH]
    t1hi = theta_1[:, _KH:]

    mesh = plsc.VectorSubcoreMesh(core_axis_name="c", subcore_axis_name="s")
    out0t, out1 = pl.kernel(
        _sc_body,
        out_type=[
            jax.ShapeDtypeStruct((4 * _NB * 8, _SUB), jnp.float32),
            jax.ShapeDtypeStruct((_N, _K), jnp.float32),
        ],
        mesh=mesh,
        compiler_params=pltpu.CompilerParams(
            use_tc_tiling_on_sc=False, needs_layout_passes=False),
        scratch_types=[
            pltpu.VMEM((2, _CH, 2, _SUB), jnp.int32),
            pltpu.VMEM((2 * _RS, _PITCH), jnp.float32),
            pltpu.VMEM((2, _CH * _SUB, _KH), jnp.float32),
            pltpu.VMEM((2, _CH * _SUB, _KH), jnp.float32),
            pltpu.VMEM_SHARED((_N, _KH), jnp.float32),
            pltpu.SemaphoreType.DMA,
            pltpu.SemaphoreType.DMA,
            pltpu.SemaphoreType.DMA,
            pltpu.SemaphoreType.DMA,
            pltpu.SemaphoreType.DMA,
        ],
    )(t0t, t1lo, t1hi, opf_t)

    out0 = jnp.transpose(out0t.reshape(4, _NB, 8, _SUB),
                         (1, 3, 0, 2)).reshape(_E, _K)
    return out0, out1
